# baseline (speedup 1.0000x reference)
import functools

import jax
import jax.numpy as jnp
from jax import lax
from jax.experimental import pallas as pl
from jax.experimental.pallas import tpu as pltpu
from jax.experimental.pallas import tpu_sc as plsc

NUM_HEADS = 32
NPOS = 64 * 64
LANES = 16
NGROUPS = NPOS // 64


def _sc_gather(table, rank_p, file_p):
    mesh = plsc.VectorSubcoreMesh(core_axis_name="c", subcore_axis_name="s")

    @functools.partial(
        pl.kernel,
        mesh=mesh,
        out_type=jax.ShapeDtypeStruct((NUM_HEADS, NPOS), jnp.float32),
        scratch_types=[
            pltpu.VMEM((15, 15), jnp.float32),
            pltpu.VMEM((NPOS // 4,), jnp.int32),
            pltpu.VMEM((NPOS // 4,), jnp.int32),
            pltpu.VMEM((NPOS,), jnp.float32),
            pltpu.SemaphoreType.DMA,
            pltpu.SemaphoreType.DMA,
            pltpu.SemaphoreType.DMA,
            pltpu.SemaphoreType.DMA,
        ],
        compiler_params=pltpu.CompilerParams(needs_layout_passes=False),
    )
    def run(table_hbm, rank_hbm, file_hbm, out_hbm,
            table_v, rank_v, file_v, out_v, sem_t, sem_r, sem_f, sem_o):
        wid = lax.axis_index("s") * 2 + lax.axis_index("c")
        ct = pltpu.async_copy(table_hbm.at[wid], table_v, sem_t)
        cr = pltpu.async_copy(rank_hbm, rank_v, sem_r)
        cf = pltpu.async_copy(file_hbm, file_v, sem_f)
        ct.wait()
        cr.wait()
        cf.wait()

        lane4 = lax.iota(jnp.int32, LANES) * 4
        chunk = NPOS // 4
        out_copies = []
        for g in range(NGROUPS):
            rw = rank_v[pl.ds(g * LANES, LANES)]
            fw = file_v[pl.ds(g * LANES, LANES)]
            pos = lane4 + g * 64
            for j in range(4):
                rb = lax.shift_right_logical(rw, 8 * j) & 0xFF
                fb = lax.shift_right_logical(fw, 8 * j) & 0xFF
                plsc.store_scatter(out_v, [pos + j],
                                   plsc.load_gather(table_v, [rb, fb]))
            if (g + 1) % (NGROUPS // 4) == 0:
                c = (g + 1) // (NGROUPS // 4) - 1
                out_copies.append(pltpu.async_copy(
                    out_v.at[pl.ds(c * chunk, chunk)],
                    out_hbm.at[wid].at[pl.ds(c * chunk, chunk)], sem_o))
        for cp in out_copies:
            cp.wait()

    return run(table, rank_p, file_p)


def _pack_u8(idx):
    v = idx.reshape(NPOS // 4, 4).astype(jnp.uint8)
    return lax.bitcast_convert_type(v, jnp.int32)


def kernel(bias_table, rank_idx, file_idx):
    out = _sc_gather(bias_table, _pack_u8(rank_idx), _pack_u8(file_idx))
    return out.reshape(NUM_HEADS, 64, 64)


# speedup vs baseline: 1.1104x; 1.1104x over previous
import functools

import jax
import jax.numpy as jnp
from jax import lax
from jax.experimental import pallas as pl
from jax.experimental.pallas import tpu as pltpu
from jax.experimental.pallas import tpu_sc as plsc

NUM_HEADS = 32
NPOS = 64 * 64
LANES = 16
NGROUPS = NPOS // 64


def _sc_gather(table, rank_p, file_p):
    mesh = plsc.VectorSubcoreMesh(core_axis_name="c", subcore_axis_name="s")

    @functools.partial(
        pl.kernel,
        mesh=mesh,
        out_type=jax.ShapeDtypeStruct((NUM_HEADS, NPOS), jnp.float32),
        scratch_types=[
            pltpu.VMEM((15, 15), jnp.float32),
            pltpu.VMEM((NPOS // 4,), jnp.int32),
            pltpu.VMEM((NPOS // 4,), jnp.int32),
            pltpu.VMEM((NPOS,), jnp.float32),
            pltpu.SemaphoreType.DMA,
            pltpu.SemaphoreType.DMA,
            pltpu.SemaphoreType.DMA,
        ],
        compiler_params=pltpu.CompilerParams(needs_layout_passes=False),
    )
    def run(table_hbm, rank_hbm, file_hbm, out_hbm,
            table_v, rank_v, file_v, out_v, sem_t, sem_r, sem_f):
        wid = lax.axis_index("s") * 2 + lax.axis_index("c")
        ct = pltpu.async_copy(table_hbm.at[wid], table_v, sem_t)
        cr = pltpu.async_copy(rank_hbm, rank_v, sem_r)
        cf = pltpu.async_copy(file_hbm, file_v, sem_f)
        ct.wait()
        cr.wait()
        cf.wait()

        for g in range(NGROUPS):
            rw = rank_v[pl.ds(g * LANES, LANES)]
            fw = file_v[pl.ds(g * LANES, LANES)]
            for j in range(4):
                rb = lax.shift_right_logical(rw, 8 * j) & 0xFF
                fb = lax.shift_right_logical(fw, 8 * j) & 0xFF
                out_v[pl.ds(g * 64 + j * LANES, LANES)] = (
                    plsc.load_gather(table_v, [rb, fb]))

        pltpu.sync_copy(out_v, out_hbm.at[wid])

    return run(table, rank_p, file_p)


def _pack_words(idx):
    v = idx.reshape(NGROUPS, 4, LANES)
    w = v[:, 0] | (v[:, 1] << 8) | (v[:, 2] << 16) | (v[:, 3] << 24)
    return w.reshape(NPOS // 4)


def kernel(bias_table, rank_idx, file_idx):
    out = _sc_gather(bias_table, _pack_words(rank_idx), _pack_words(file_idx))
    return out.reshape(NUM_HEADS, 64, 64)


# speedup vs baseline: 1.1753x; 1.0585x over previous
import functools

import jax
import jax.numpy as jnp
from jax import lax
from jax.experimental import pallas as pl
from jax.experimental.pallas import tpu as pltpu
from jax.experimental.pallas import tpu_sc as plsc

NUM_HEADS = 32
NPOS = 64 * 64
LANES = 16
NGROUPS = NPOS // 64


def _sc_gather(table, packed):
    mesh = plsc.VectorSubcoreMesh(core_axis_name="c", subcore_axis_name="s")

    @functools.partial(
        pl.kernel,
        mesh=mesh,
        out_type=jax.ShapeDtypeStruct((NUM_HEADS, 64, 64), jnp.float32),
        scratch_types=[
            pltpu.VMEM((15, 15), jnp.float32),
            pltpu.VMEM((NGROUPS, LANES), jnp.int32),
            pltpu.VMEM((64, 64), jnp.float32),
            pltpu.SemaphoreType.DMA,
            pltpu.SemaphoreType.DMA,
        ],
        compiler_params=pltpu.CompilerParams(needs_layout_passes=False),
    )
    def run(table_hbm, packed_hbm, out_hbm, table_v, pk_v, out_v, sem_t, sem_p):
        wid = lax.axis_index("s") * 2 + lax.axis_index("c")
        ct = pltpu.async_copy(table_hbm.at[wid], table_v, sem_t)
        cp = pltpu.async_copy(packed_hbm, pk_v, sem_p)
        ct.wait()
        cp.wait()

        for g in range(NGROUPS):
            w = pk_v[g]
            for j in range(4):
                b = lax.shift_right_logical(w, 8 * j) & 0xFF
                rb = b & 0xF
                fb = lax.shift_right_logical(b, 4)
                out_v[g, pl.ds(j * LANES, LANES)] = (
                    plsc.load_gather(table_v, [rb, fb]))

        pltpu.sync_copy(out_v, out_hbm.at[wid])

    return run(table, packed)


def _pack_nibbles(rank_idx, file_idx):
    m = (rank_idx | (file_idx << 4)).reshape(NGROUPS, 4, LANES)
    return m[:, 0] | (m[:, 1] << 8) | (m[:, 2] << 16) | (m[:, 3] << 24)


def kernel(bias_table, rank_idx, file_idx):
    return _sc_gather(bias_table, _pack_nibbles(rank_idx, file_idx))


# speedup vs baseline: 1.2494x; 1.0630x over previous
import functools

import jax
import jax.numpy as jnp
from jax import lax
from jax.experimental import pallas as pl
from jax.experimental.pallas import tpu as pltpu
from jax.experimental.pallas import tpu_sc as plsc

NUM_HEADS = 32
NPOS = 64 * 64
LANES = 16
NGROUPS = NPOS // 64


def _sc_gather(table, packed):
    mesh = plsc.VectorSubcoreMesh(core_axis_name="c", subcore_axis_name="s")

    @functools.partial(
        pl.kernel,
        mesh=mesh,
        out_type=jax.ShapeDtypeStruct((NUM_HEADS, 64, 64), jnp.float32),
        scratch_types=[
            pltpu.VMEM((15, 15), jnp.float32),
            pltpu.VMEM((NPOS // 4,), jnp.int32),
            pltpu.VMEM((64, 64), jnp.float32),
            pltpu.SemaphoreType.DMA,
            pltpu.SemaphoreType.DMA,
        ],
        compiler_params=pltpu.CompilerParams(needs_layout_passes=False),
    )
    def run(table_hbm, packed_hbm, out_hbm, table_v, pk_v, out_v, sem_t, sem_p):
        wid = lax.axis_index("s") * 2 + lax.axis_index("c")
        ct = pltpu.async_copy(table_hbm.at[wid], table_v, sem_t)
        cp = pltpu.async_copy(packed_hbm, pk_v, sem_p)
        ct.wait()
        cp.wait()

        for g in range(NGROUPS):
            w = pk_v[pl.ds(g * LANES, LANES)]
            for j in range(4):
                b = lax.shift_right_logical(w, 8 * j) & 0xFF
                rb = b & 0xF
                fb = lax.shift_right_logical(b, 4)
                out_v[g, pl.ds(j * LANES, LANES)] = (
                    plsc.load_gather(table_v, [rb, fb]))

        pltpu.sync_copy(out_v, out_hbm.at[wid])

    return run(table, packed)


def _pack_nibbles(rank_idx, file_idx):
    m = (rank_idx | (file_idx << 4)).reshape(NGROUPS, 4, LANES)
    w = m[:, 0] | (m[:, 1] << 8) | (m[:, 2] << 16) | (m[:, 3] << 24)
    return w.reshape(NPOS // 4)


def kernel(bias_table, rank_idx, file_idx):
    return _sc_gather(bias_table, _pack_nibbles(rank_idx, file_idx))
